# baseline (speedup 1.0000x reference)
import jax
import jax.numpy as jnp
from jax.experimental import pallas as pl
from jax.experimental.pallas import tpu as pltpu

_B = 16
_M = 255
_SHAPES = [(256, 4096), (512, 1024), (1024, 256)]


def _make_kernel():
    mesh = pltpu.create_tensorcore_mesh("core")
    num_cores = mesh.devices.shape[0]
    per_core = _B // num_cores

    scratch = []
    for c, hw in _SHAPES:
        scratch.append(pltpu.VMEM((2, c, hw), jnp.float32))
        scratch.append(pltpu.VMEM((2, _M, hw), jnp.float32))
        scratch.append(pltpu.VMEM((_M, c), jnp.float32))
        scratch.append(pltpu.VMEM((_M, 1), jnp.float32))
        scratch.append(pltpu.SemaphoreType.DMA((2,)))
        scratch.append(pltpu.SemaphoreType.DMA((2,)))
    scratch.append(pltpu.SemaphoreType.DMA((6,)))

    def body(x0, x1, x2, w0, b0, w1, b1, w2, b2, o0, o1, o2, *scr):
        (xb0, ob0, wb0, bb0, si0, so0,
         xb1, ob1, wb1, bb1, si1, so1,
         xb2, ob2, wb2, bb2, si2, so2, swb) = scr
        core = jax.lax.axis_index("core")
        base = core * per_core

        wcps = [
            pltpu.make_async_copy(w0, wb0, swb.at[0]),
            pltpu.make_async_copy(b0, bb0, swb.at[1]),
            pltpu.make_async_copy(w1, wb1, swb.at[2]),
            pltpu.make_async_copy(b1, bb1, swb.at[3]),
            pltpu.make_async_copy(w2, wb2, swb.at[4]),
            pltpu.make_async_copy(b2, bb2, swb.at[5]),
        ]
        for cp in wcps:
            cp.start()
        for cp in wcps:
            cp.wait()

        def run_scale(x_hbm, o_hbm, xb, ob, wb, bb, sin, sout):
            def in_cp(i, slot):
                return pltpu.make_async_copy(
                    x_hbm.at[base + i], xb.at[slot], sin.at[slot])

            def out_cp(i, slot):
                return pltpu.make_async_copy(
                    ob.at[slot], o_hbm.at[base + i], sout.at[slot])

            in_cp(0, 0).start()
            for i in range(per_core):
                slot = i % 2
                if i + 1 < per_core:
                    in_cp(i + 1, slot ^ 1).start()
                in_cp(i, slot).wait()
                if i >= 2:
                    out_cp(i - 2, slot).wait()
                acc = jnp.dot(wb[...], xb[slot],
                              preferred_element_type=jnp.float32)
                ob[slot] = acc + bb[...]
                out_cp(i, slot).start()
            for i in range(max(per_core - 2, 0), per_core):
                out_cp(i, i % 2).wait()

        run_scale(x0, o0, xb0, ob0, wb0, bb0, si0, so0)
        run_scale(x1, o1, xb1, ob1, wb1, bb1, si1, so1)
        run_scale(x2, o2, xb2, ob2, wb2, bb2, si2, so2)

    out_type = tuple(
        jax.ShapeDtypeStruct((_B, _M, hw), jnp.float32) for _, hw in _SHAPES
    )
    return pl.kernel(body, out_type=out_type, mesh=mesh,
                     scratch_types=tuple(scratch))


def kernel(feat0, feat1, feat2, W0, b0, W1, b1, W2, b2):
    xs = [f.reshape(_B, c, hw) for f, (c, hw) in
          zip((feat0, feat1, feat2), _SHAPES)]
    ws = [W.reshape(_M, c) for W, (c, _) in zip((W0, W1, W2), _SHAPES)]
    bs = [b.reshape(_M, 1) for b in (b0, b1, b2)]
    k = _make_kernel()
    o0, o1, o2 = k(xs[0], xs[1], xs[2],
                   ws[0], bs[0], ws[1], bs[1], ws[2], bs[2])
    return (
        o0.reshape(_B, _M, 64, 64),
        o1.reshape(_B, _M, 32, 32),
        o2.reshape(_B, _M, 16, 16),
    )


# speedup vs baseline: 1.0231x; 1.0231x over previous
import jax
import jax.numpy as jnp
from jax.experimental import pallas as pl
from jax.experimental.pallas import tpu as pltpu


def _head_body(x_ref, w_ref, b_ref, o_ref):
    acc = jnp.dot(w_ref[...], x_ref[0], preferred_element_type=jnp.float32)
    o_ref[...] = (acc + b_ref[...]).astype(jnp.bfloat16)[None]


def _head_matmul(x, w, b):
    B, C, HW = x.shape
    M = w.shape[0]
    return pl.pallas_call(
        _head_body,
        grid=(B,),
        in_specs=[
            pl.BlockSpec((1, C, HW), lambda i: (i, 0, 0)),
            pl.BlockSpec((M, C), lambda i: (0, 0)),
            pl.BlockSpec((M, 1), lambda i: (0, 0)),
        ],
        out_specs=pl.BlockSpec((1, M, HW), lambda i: (i, 0, 0)),
        out_shape=jax.ShapeDtypeStruct((B, M, HW), jnp.bfloat16),
        compiler_params=pltpu.CompilerParams(
            dimension_semantics=("parallel",),
        ),
    )(x, w, b)


def _scale(feat, W, b, H):
    B, C, HW = feat.shape[0], feat.shape[1], feat.shape[2] * feat.shape[3]
    M = W.shape[0]
    x = feat.reshape(B, C, HW).astype(jnp.bfloat16)
    w2 = W.reshape(M, C).astype(jnp.bfloat16)
    out = _head_matmul(x, w2, b.reshape(M, 1))
    return out.astype(jnp.float32).reshape(B, M, H, H)


def kernel(feat0, feat1, feat2, W0, b0, W1, b1, W2, b2):
    return (
        _scale(feat0, W0, b0, 64),
        _scale(feat1, W1, b1, 32),
        _scale(feat2, W2, b2, 16),
    )
